# baseline (speedup 1.0000x reference)
import functools

import jax
import jax.numpy as jnp
from jax import lax
from jax.experimental import pallas as pl
from jax.experimental.pallas import tpu as pltpu
from jax.experimental.pallas import tpu_sc as plsc

_B, _H, _W = 256, 64, 1024
_N = 50000
_NW = 32
_BPW = _B // _NW
_CHUNK = 2000
_NCHUNK = _N // _CHUNK
_NPAIR = (_NCHUNK - 1) // 2
_NQUAD = _NCHUNK // 4
_VPC = _CHUNK // 16


def _make_kernel():
    mesh = plsc.VectorSubcoreMesh(core_axis_name="c", subcore_axis_name="s")

    @functools.partial(
        pl.kernel,
        mesh=mesh,
        out_type=jax.ShapeDtypeStruct((_B * _N,), jnp.float32),
        compiler_params=pltpu.CompilerParams(needs_layout_passes=False),
        scratch_types=[
            pltpu.VMEM((_N,), jnp.int32),
            pltpu.VMEM((_H, _W), jnp.float32),
            pltpu.VMEM((_CHUNK,), jnp.float32),
            pltpu.VMEM((_CHUNK,), jnp.float32),
            pltpu.VMEM((_CHUNK,), jnp.float32),
            pltpu.VMEM((_CHUNK,), jnp.float32),
            pltpu.VMEM((_CHUNK,), jnp.float32),
            pltpu.VMEM((_CHUNK,), jnp.float32),
            pltpu.SemaphoreType.DMA,
            pltpu.SemaphoreType.DMA,
            pltpu.SemaphoreType.DMA,
            pltpu.SemaphoreType.DMA,
            pltpu.SemaphoreType.DMA,
            pltpu.SemaphoreType.DMA,
        ],
    )
    def gather_kernel(x_hbm, y_hbm, data_hbm, out_hbm,
                      idx_v, row_v, o0_v, o1_v, o2_v, o3_v, xb_v, yb_v,
                      sem0, sem1, sem2, sem3, semb, semr):
        wid = lax.axis_index("s") * 2 + lax.axis_index("c")
        b0 = wid * _BPW
        obufs = (o0_v, o1_v, o2_v, o3_v)
        osems = (sem0, sem1, sem2, sem3)

        pltpu.async_copy(data_hbm.at[b0], row_v, semr)

        def start_load(c, xbuf, ybuf, semx, semy):
            base = pl.multiple_of(c * _CHUNK, _CHUNK)
            pltpu.async_copy(x_hbm.at[pl.ds(base, _CHUNK)], xbuf, semx)
            pltpu.async_copy(y_hbm.at[pl.ds(base, _CHUNK)], ybuf, semy)

        def wait_load(c, xbuf, ybuf, semx, semy):
            base = pl.multiple_of(c * _CHUNK, _CHUNK)
            pltpu.make_async_copy(x_hbm.at[pl.ds(base, _CHUNK)], xbuf, semx).wait()
            pltpu.make_async_copy(y_hbm.at[pl.ds(base, _CHUNK)], ybuf, semy).wait()

        def pack_chunk(c, xbuf, ybuf):
            base = pl.multiple_of(c * _CHUNK, _CHUNK)

            @plsc.parallel_loop(0, _VPC, unroll=8)
            def _build(j):
                xv = plsc.bitcast(xbuf[pl.ds(j * 16, 16)], jnp.int32)
                yv = plsc.bitcast(ybuf[pl.ds(j * 16, 16)], jnp.int32)
                idx_v[pl.ds(base + j * 16, 16)] = (xv << 10) | yv

        start_load(0, o2_v, o3_v, sem2, sem3)

        def build_pair(t, carry):
            c0 = t * 2
            wait_load(c0, o2_v, o3_v, sem2, sem3)
            start_load(c0 + 1, xb_v, yb_v, semb, semb)
            pack_chunk(c0, o2_v, o3_v)
            wait_load(c0 + 1, xb_v, yb_v, semb, semb)
            start_load(c0 + 2, o2_v, o3_v, sem2, sem3)
            pack_chunk(c0 + 1, xb_v, yb_v)
            return carry

        lax.fori_loop(0, _NPAIR, build_pair, None)
        wait_load(_NCHUNK - 1, o2_v, o3_v, sem2, sem3)
        pack_chunk(_NCHUNK - 1, o2_v, o3_v)

        def gather_chunk(buf, cbase):
            @plsc.parallel_loop(0, _VPC, unroll=8)
            def _gather(j):
                pk = idx_v[pl.ds(cbase + j * 16, 16)]
                ix = pk >> 10
                iy = pk & 1023
                buf[pl.ds(j * 16, 16)] = plsc.load_gather(row_v, [ix, iy])

        def do_batch(i, carry):
            b = b0 + i
            out_off = pl.multiple_of(b * _N, 8)
            pltpu.make_async_copy(data_hbm.at[b], row_v, semr).wait()

            def quad(t, carry2):
                qbase = pl.multiple_of(t * 4 * _CHUNK, _CHUNK)
                for k in range(4):
                    base = qbase + k * _CHUNK
                    dst = out_hbm.at[pl.ds(out_off + base, _CHUNK)]
                    buf, sem = obufs[k], osems[k]

                    @pl.when(t > 0)
                    def _wait(buf=buf, dst=dst, sem=sem):
                        pltpu.make_async_copy(buf, dst, sem).wait()

                    gather_chunk(buf, base)
                    pltpu.async_copy(buf, dst, sem)
                return carry2

            lax.fori_loop(0, _NQUAD, quad, None)

            tail = pl.multiple_of(4 * _NQUAD * _CHUNK, _CHUNK)
            dst_t = out_hbm.at[pl.ds(out_off + tail, _CHUNK)]
            pltpu.make_async_copy(o0_v, dst_t, sem0).wait()
            gather_chunk(o0_v, tail)
            pltpu.async_copy(o0_v, dst_t, sem0)

            @pl.when(i + 1 < _BPW)
            def _prefetch():
                pltpu.async_copy(data_hbm.at[b + 1], row_v, semr)

            pltpu.make_async_copy(o1_v, dst_t, sem1).wait()
            pltpu.make_async_copy(o2_v, dst_t, sem2).wait()
            pltpu.make_async_copy(o3_v, dst_t, sem3).wait()
            pltpu.make_async_copy(o0_v, dst_t, sem0).wait()
            return carry

        lax.fori_loop(0, _BPW, do_batch, None)

    return gather_kernel


_gather = _make_kernel()


def kernel(data, x_indices, y_indices):
    x = lax.bitcast_convert_type(x_indices.astype(jnp.int32), jnp.float32)
    y = lax.bitcast_convert_type(y_indices.astype(jnp.int32), jnp.float32)
    out = _gather(x, y, data)
    return out.reshape(_B, _N)
